# baseline (speedup 1.0000x reference)
import functools

import jax
import jax.numpy as jnp
from jax import lax
from jax.experimental import pallas as pl
from jax.experimental.pallas import tpu as pltpu
from jax.experimental.pallas import tpu_sc as plsc

_B = 16384
_F = 26
_V = 2068
_D = 128
_TW = 144
_NC = 2
_NS = 16
_NW = _NC * _NS
_RPW = _B // _NW
_CH = 2
_NCH = _RPW // _CH
_IW = _CH * _F


def _prep_body(emb1_ref, emb2_ref, taug_ref):
    e2 = emb2_ref[...]
    rowsq = jnp.sum(e2 * e2, axis=1, keepdims=True)
    comb = emb1_ref[...] - 0.5 * rowsq
    pad = jnp.zeros((_V, _TW - _D - 1), jnp.float32)
    taug_ref[...] = jnp.concatenate([e2, comb, pad], axis=1)


def _sc_body(taug, idxp, d0, d1, d2, wb, out,
             idx_v, buf0, buf1, buf2, buf3, buf4, buf5, buf6, buf7,
             d0_v, d1_v, d2_v, wb_v, out_v,
             sem0, sem1, sem2, sem3, sem4, sem5, sem6, sem7):
    wid = lax.axis_index("s") * _NC + lax.axis_index("c")

    pltpu.sync_copy(idxp.at[pl.ds(wid * _NCH, _NCH)], idx_v)
    pltpu.sync_copy(d0.at[pl.ds(wid * _RPW, _RPW)], d0_v)
    pltpu.sync_copy(d1.at[pl.ds(wid * _RPW, _RPW)], d1_v)
    pltpu.sync_copy(d2.at[pl.ds(wid * _RPW, _RPW)], d2_v)
    pltpu.sync_copy(wb, wb_v)

    def start(j, buf, sem):
        pltpu.make_async_copy(taug.at[idx_v.at[j]], buf, sem).start()

    def wait(j, buf, sem):
        pltpu.make_async_copy(taug.at[idx_v.at[j]], buf, sem).wait()

    lanes = lax.iota(jnp.int32, 16)

    def compute(buf, yv, lane_base):
        nv = _TW // 16
        for rr in range(_CH):
            base = rr * _F

            def fbody(f, accs):
                return tuple(a + buf[base + f, pl.ds(v * 16, 16)]
                             for v, a in enumerate(accs))

            accs = lax.fori_loop(
                1, _F, fbody,
                tuple(buf[base, pl.ds(v * 16, 16)] for v in range(nv)))
            sq = accs[0] * accs[0]
            for v in range(1, _D // 16):
                sq = sq + accs[v] * accs[v]
            s = jnp.sum(0.5 * sq + accs[_D // 16])
            yv = jnp.where(lanes == lane_base + rr, s, yv)
        return yv

    bufs = [buf0, buf1, buf2, buf3, buf4, buf5, buf6, buf7]
    sems = [sem0, sem1, sem2, sem3, sem4, sem5, sem6, sem7]
    for p in range(8):
        start(p, bufs[p], sems[p])

    w0 = wb_v[0, pl.ds(0, 16)]
    w1 = wb_v[1, pl.ds(0, 16)]
    w2 = wb_v[2, pl.ds(0, 16)]
    bb = wb_v[3, pl.ds(0, 16)]

    def loop_body(i, carry):
        j0 = 8 * i
        yv = jnp.zeros((16,), jnp.float32)
        for p in range(8):
            wait(j0 + p, bufs[p], sems[p])
            yv = compute(bufs[p], yv, 2 * p)
            start(jnp.minimum(j0 + p + 8, _NCH - 1), bufs[p], sems[p])

        sl = pl.ds(i * 16, 16)
        t = w0 * d0_v[sl] + w1 * d1_v[sl] + w2 * d2_v[sl] + bb
        sfull = yv + t
        out_v[sl] = 1.0 / (1.0 + jnp.exp(-sfull))
        return carry

    lax.fori_loop(0, _NCH // 8, loop_body, 0)
    for p in range(8):
        wait(_NCH - 1, bufs[p], sems[p])
    pltpu.sync_copy(out_v, out.at[pl.ds(wid * _RPW, _RPW)])


@jax.jit
def kernel(dense_input, sparse_input, emb1, emb2, W1, b1):
    taug = pl.pallas_call(
        _prep_body,
        out_shape=jax.ShapeDtypeStruct((_V, _TW), jnp.float32),
    )(emb1, emb2)

    idxp = sparse_input.astype(jnp.int32).reshape(_B // _CH, _IW)
    d0 = dense_input[:, 0]
    d1 = dense_input[:, 1]
    d2 = dense_input[:, 2]
    wb = jnp.broadcast_to(
        jnp.concatenate([W1.reshape(3), b1.reshape(1)]).astype(jnp.float32)[:, None],
        (4, 16),
    )

    mesh = plsc.VectorSubcoreMesh(core_axis_name="c", subcore_axis_name="s")
    sc = functools.partial(
        pl.kernel,
        out_type=jax.ShapeDtypeStruct((_B,), jnp.float32),
        mesh=mesh,
        compiler_params=pltpu.CompilerParams(
            needs_layout_passes=False, use_tc_tiling_on_sc=False
        ),
        scratch_types=[
            pltpu.VMEM((_NCH, _IW), jnp.int32),
            pltpu.VMEM((_IW, _TW), jnp.float32),
            pltpu.VMEM((_IW, _TW), jnp.float32),
            pltpu.VMEM((_IW, _TW), jnp.float32),
            pltpu.VMEM((_IW, _TW), jnp.float32),
            pltpu.VMEM((_IW, _TW), jnp.float32),
            pltpu.VMEM((_IW, _TW), jnp.float32),
            pltpu.VMEM((_IW, _TW), jnp.float32),
            pltpu.VMEM((_IW, _TW), jnp.float32),
            pltpu.VMEM((_RPW,), jnp.float32),
            pltpu.VMEM((_RPW,), jnp.float32),
            pltpu.VMEM((_RPW,), jnp.float32),
            pltpu.VMEM((4, 16), jnp.float32),
            pltpu.VMEM((_RPW,), jnp.float32),
            pltpu.SemaphoreType.DMA,
            pltpu.SemaphoreType.DMA,
            pltpu.SemaphoreType.DMA,
            pltpu.SemaphoreType.DMA,
            pltpu.SemaphoreType.DMA,
            pltpu.SemaphoreType.DMA,
            pltpu.SemaphoreType.DMA,
            pltpu.SemaphoreType.DMA,
        ],
    )(_sc_body)
    y = sc(taug, idxp, d0, d1, d2, wb)
    return y.reshape(_B, 1)


# speedup vs baseline: 1.3539x; 1.3539x over previous
import functools

import jax
import jax.numpy as jnp
from jax import lax
from jax.experimental import pallas as pl
from jax.experimental.pallas import tpu as pltpu
from jax.experimental.pallas import tpu_sc as plsc

_B = 16384
_F = 26
_V = 2068
_D = 128
_TW = 144
_NC = 2
_NS = 16
_NW = _NC * _NS
_RPW = _B // _NW
_CH = 4
_NCH = _RPW // _CH
_IW = _CH * _F


_VP = 2080


def _prep_body(emb1_ref, emb2_ref, taug_ref):
    e2 = emb2_ref[...]
    rowsq = jnp.sum(e2 * e2, axis=1, keepdims=True)
    comb = emb1_ref[...] - 0.5 * rowsq
    pad = jnp.zeros((_V, _TW - _D - 1), jnp.float32)
    t = jnp.concatenate([e2, comb, pad], axis=1)
    taug_ref[...] = jnp.concatenate(
        [t, jnp.zeros((_VP - _V, _TW), jnp.float32)], axis=0)


def _sc_body(taug, idxp, d0, d1, d2, wb, out,
             idx_v, tspm, buf0, buf1, buf2, buf3, d0_v, d1_v, d2_v, wb_v,
             out_v, sem0, sem1, sem2, sem3):
    wid = lax.axis_index("s") * _NC + lax.axis_index("c")

    sid = lax.axis_index("s")
    rows_per_sub = 130
    rbase = sid * rows_per_sub
    pltpu.sync_copy(taug.at[pl.ds(rbase, rows_per_sub)],
                    tspm.at[pl.ds(rbase, rows_per_sub)])

    plsc.subcore_barrier()

    pltpu.sync_copy(idxp.at[pl.ds(wid * _NCH, _NCH)], idx_v)
    pltpu.sync_copy(d0.at[pl.ds(wid * _RPW, _RPW)], d0_v)
    pltpu.sync_copy(d1.at[pl.ds(wid * _RPW, _RPW)], d1_v)
    pltpu.sync_copy(d2.at[pl.ds(wid * _RPW, _RPW)], d2_v)
    pltpu.sync_copy(wb, wb_v)

    def start(j, buf, sem):
        pltpu.make_async_copy(tspm.at[idx_v.at[j]], buf, sem).start()

    def wait(j, buf, sem):
        pltpu.make_async_copy(tspm.at[idx_v.at[j]], buf, sem).wait()

    lanes = lax.iota(jnp.int32, 16)

    def compute(buf, yv, lane_base):
        nv = _TW // 16
        for rr in range(_CH):
            base = rr * _F

            def fbody(f, accs):
                return tuple(a + buf[base + f, pl.ds(v * 16, 16)]
                             for v, a in enumerate(accs))

            accs = lax.fori_loop(
                1, _F, fbody,
                tuple(buf[base, pl.ds(v * 16, 16)] for v in range(nv)))
            sq = accs[0] * accs[0]
            for v in range(1, _D // 16):
                sq = sq + accs[v] * accs[v]
            s = jnp.sum(0.5 * sq + accs[_D // 16])
            yv = jnp.where(lanes == lane_base + rr, s, yv)
        return yv

    bufs = [buf0, buf1, buf2, buf3]
    sems = [sem0, sem1, sem2, sem3]
    for p in range(4):
        start(p, bufs[p], sems[p])

    w0 = wb_v[0, pl.ds(0, 16)]
    w1 = wb_v[1, pl.ds(0, 16)]
    w2 = wb_v[2, pl.ds(0, 16)]
    bb = wb_v[3, pl.ds(0, 16)]

    def loop_body(i, carry):
        j0 = 4 * i
        yv = jnp.zeros((16,), jnp.float32)
        for p in range(4):
            wait(j0 + p, bufs[p], sems[p])
            yv = compute(bufs[p], yv, 4 * p)
            start(jnp.minimum(j0 + p + 4, _NCH - 1), bufs[p], sems[p])

        sl = pl.ds(i * 16, 16)
        t = w0 * d0_v[sl] + w1 * d1_v[sl] + w2 * d2_v[sl] + bb
        sfull = yv + t
        out_v[sl] = 1.0 / (1.0 + jnp.exp(-sfull))
        return carry

    lax.fori_loop(0, _NCH // 4, loop_body, 0)
    for p in range(4):
        wait(_NCH - 1, bufs[p], sems[p])
    pltpu.sync_copy(out_v, out.at[pl.ds(wid * _RPW, _RPW)])


@jax.jit
def kernel(dense_input, sparse_input, emb1, emb2, W1, b1):
    taug = pl.pallas_call(
        _prep_body,
        out_shape=jax.ShapeDtypeStruct((_VP, _TW), jnp.float32),
    )(emb1, emb2)

    idxp = sparse_input.astype(jnp.int32).reshape(_B // _CH, _IW)
    d0 = dense_input[:, 0]
    d1 = dense_input[:, 1]
    d2 = dense_input[:, 2]
    wb = jnp.broadcast_to(
        jnp.concatenate([W1.reshape(3), b1.reshape(1)]).astype(jnp.float32)[:, None],
        (4, 16),
    )

    mesh = plsc.VectorSubcoreMesh(core_axis_name="c", subcore_axis_name="s")
    sc = functools.partial(
        pl.kernel,
        out_type=jax.ShapeDtypeStruct((_B,), jnp.float32),
        mesh=mesh,
        compiler_params=pltpu.CompilerParams(
            needs_layout_passes=False, use_tc_tiling_on_sc=False
        ),
        scratch_types=[
            pltpu.VMEM((_NCH, _IW), jnp.int32),
            pltpu.VMEM_SHARED((2080, _TW), jnp.float32),
            pltpu.VMEM((_IW, _TW), jnp.float32),
            pltpu.VMEM((_IW, _TW), jnp.float32),
            pltpu.VMEM((_IW, _TW), jnp.float32),
            pltpu.VMEM((_IW, _TW), jnp.float32),
            pltpu.VMEM((_RPW,), jnp.float32),
            pltpu.VMEM((_RPW,), jnp.float32),
            pltpu.VMEM((_RPW,), jnp.float32),
            pltpu.VMEM((4, 16), jnp.float32),
            pltpu.VMEM((_RPW,), jnp.float32),
            pltpu.SemaphoreType.DMA,
            pltpu.SemaphoreType.DMA,
            pltpu.SemaphoreType.DMA,
            pltpu.SemaphoreType.DMA,
        ],
    )(_sc_body)
    y = sc(taug, idxp, d0, d1, d2, wb)
    return y.reshape(_B, 1)


# speedup vs baseline: 1.4291x; 1.0556x over previous
import functools

import jax
import jax.numpy as jnp
from jax import lax
from jax.experimental import pallas as pl
from jax.experimental.pallas import tpu as pltpu
from jax.experimental.pallas import tpu_sc as plsc

_B = 16384
_F = 26
_V = 2068
_D = 128
_TW = 144
_NC = 2
_NS = 16
_NW = _NC * _NS
_RPW = _B // _NW
_CH = 4
_NCH = _RPW // _CH
_IW = _CH * _F


_VP = 2080


def _prep_body(emb1_ref, emb2_ref, taug_ref):
    e2 = emb2_ref[...]
    rowsq = jnp.sum(e2 * e2, axis=1, keepdims=True)
    comb = emb1_ref[...] - 0.5 * rowsq
    pad = jnp.zeros((_V, _TW - _D - 1), jnp.float32)
    t = jnp.concatenate([e2, comb, pad], axis=1)
    taug_ref[...] = jnp.concatenate(
        [t, jnp.zeros((_VP - _V, _TW), jnp.float32)], axis=0)


def _sc_body(taug, idxp, d0, d1, d2, wb, out,
             idx_v, tspm, buf0, buf1, buf2, buf3, d0_v, d1_v, d2_v, wb_v,
             out_v, sem0, sem1, sem2, sem3):
    wid = lax.axis_index("s") * _NC + lax.axis_index("c")

    sid = lax.axis_index("s")
    rows_per_sub = 130
    rbase = sid * rows_per_sub
    pltpu.sync_copy(taug.at[pl.ds(rbase, rows_per_sub)],
                    tspm.at[pl.ds(rbase, rows_per_sub)])

    plsc.subcore_barrier()

    pltpu.sync_copy(idxp.at[pl.ds(wid * _NCH, _NCH)], idx_v)
    pltpu.sync_copy(d0.at[pl.ds(wid * _RPW, _RPW)], d0_v)
    pltpu.sync_copy(d1.at[pl.ds(wid * _RPW, _RPW)], d1_v)
    pltpu.sync_copy(d2.at[pl.ds(wid * _RPW, _RPW)], d2_v)
    pltpu.sync_copy(wb, wb_v)

    def start(j, buf, sem, src):
        pltpu.make_async_copy(src.at[idx_v.at[j]], buf, sem).start()

    def wait(j, buf, sem, src):
        pltpu.make_async_copy(src.at[idx_v.at[j]], buf, sem).wait()

    lanes = lax.iota(jnp.int32, 16)

    def compute(buf, yv, lane_base):
        nv = _TW // 16
        for rr in range(_CH):
            base = rr * _F

            def fbody(f, accs):
                return tuple(a + buf[base + f, pl.ds(v * 16, 16)]
                             for v, a in enumerate(accs))

            accs = lax.fori_loop(
                1, _F, fbody,
                tuple(buf[base, pl.ds(v * 16, 16)] for v in range(nv)))
            sq = accs[0] * accs[0]
            for v in range(1, _D // 16):
                sq = sq + accs[v] * accs[v]
            s = jnp.sum(0.5 * sq + accs[_D // 16])
            yv = jnp.where(lanes == lane_base + rr, s, yv)
        return yv

    bufs = [buf0, buf1, buf2, buf3]
    sems = [sem0, sem1, sem2, sem3]
    srcs = [tspm, taug, tspm, taug]
    for p in range(4):
        start(p, bufs[p], sems[p], srcs[p])

    w0 = wb_v[0, pl.ds(0, 16)]
    w1 = wb_v[1, pl.ds(0, 16)]
    w2 = wb_v[2, pl.ds(0, 16)]
    bb = wb_v[3, pl.ds(0, 16)]

    def loop_body(i, carry):
        j0 = 4 * i
        yv = jnp.zeros((16,), jnp.float32)
        for p in range(4):
            wait(j0 + p, bufs[p], sems[p], srcs[p])
            yv = compute(bufs[p], yv, 4 * p)
            start(jnp.minimum(j0 + p + 4, _NCH - 1), bufs[p], sems[p],
                  srcs[p])

        sl = pl.ds(i * 16, 16)
        t = w0 * d0_v[sl] + w1 * d1_v[sl] + w2 * d2_v[sl] + bb
        sfull = yv + t
        out_v[sl] = 1.0 / (1.0 + jnp.exp(-sfull))
        return carry

    lax.fori_loop(0, _NCH // 4, loop_body, 0)
    for p in range(4):
        wait(_NCH - 1, bufs[p], sems[p], srcs[p])
    pltpu.sync_copy(out_v, out.at[pl.ds(wid * _RPW, _RPW)])


@jax.jit
def kernel(dense_input, sparse_input, emb1, emb2, W1, b1):
    taug = pl.pallas_call(
        _prep_body,
        out_shape=jax.ShapeDtypeStruct((_VP, _TW), jnp.float32),
    )(emb1, emb2)

    idxp = sparse_input.astype(jnp.int32).reshape(_B // _CH, _IW)
    d0 = dense_input[:, 0]
    d1 = dense_input[:, 1]
    d2 = dense_input[:, 2]
    wb = jnp.broadcast_to(
        jnp.concatenate([W1.reshape(3), b1.reshape(1)]).astype(jnp.float32)[:, None],
        (4, 16),
    )

    mesh = plsc.VectorSubcoreMesh(core_axis_name="c", subcore_axis_name="s")
    sc = functools.partial(
        pl.kernel,
        out_type=jax.ShapeDtypeStruct((_B,), jnp.float32),
        mesh=mesh,
        compiler_params=pltpu.CompilerParams(
            needs_layout_passes=False, use_tc_tiling_on_sc=False
        ),
        scratch_types=[
            pltpu.VMEM((_NCH, _IW), jnp.int32),
            pltpu.VMEM_SHARED((2080, _TW), jnp.float32),
            pltpu.VMEM((_IW, _TW), jnp.float32),
            pltpu.VMEM((_IW, _TW), jnp.float32),
            pltpu.VMEM((_IW, _TW), jnp.float32),
            pltpu.VMEM((_IW, _TW), jnp.float32),
            pltpu.VMEM((_RPW,), jnp.float32),
            pltpu.VMEM((_RPW,), jnp.float32),
            pltpu.VMEM((_RPW,), jnp.float32),
            pltpu.VMEM((4, 16), jnp.float32),
            pltpu.VMEM((_RPW,), jnp.float32),
            pltpu.SemaphoreType.DMA,
            pltpu.SemaphoreType.DMA,
            pltpu.SemaphoreType.DMA,
            pltpu.SemaphoreType.DMA,
        ],
    )(_sc_body)
    y = sc(taug, idxp, d0, d1, d2, wb)
    return y.reshape(_B, 1)


# speedup vs baseline: 1.4431x; 1.0098x over previous
import functools

import jax
import jax.numpy as jnp
from jax import lax
from jax.experimental import pallas as pl
from jax.experimental.pallas import tpu as pltpu
from jax.experimental.pallas import tpu_sc as plsc

_B = 16384
_F = 26
_V = 2068
_D = 128
_TW = 144
_NC = 2
_NS = 16
_NW = _NC * _NS
_RPW = _B // _NW
_CH = 4
_NCH = _RPW // _CH
_IW = _CH * _F


_VP = 2080


def _prep_body(emb1_ref, emb2_ref, taug_ref):
    e2 = emb2_ref[...]
    rowsq = jnp.sum(e2 * e2, axis=1, keepdims=True)
    comb = emb1_ref[...] - 0.5 * rowsq
    pad = jnp.zeros((_V, _TW - _D - 1), jnp.float32)
    t = jnp.concatenate([e2, comb, pad], axis=1)
    taug_ref[...] = jnp.concatenate(
        [t, jnp.zeros((_VP - _V, _TW), jnp.float32)], axis=0)


def _sc_body(taug, idxp, d0, d1, d2, wb, out,
             idx_v, tspm, buf0, buf1, buf2, buf3, d0_v, d1_v, d2_v, wb_v,
             out_v, sem0, sem1, sem2, sem3):
    wid = lax.axis_index("s") * _NC + lax.axis_index("c")

    sid = lax.axis_index("s")
    rows_per_sub = 130
    rbase = sid * rows_per_sub
    pltpu.sync_copy(taug.at[pl.ds(rbase, rows_per_sub)],
                    tspm.at[pl.ds(rbase, rows_per_sub)])

    plsc.subcore_barrier()

    pltpu.sync_copy(idxp.at[pl.ds(wid * _NCH, _NCH)], idx_v)
    pltpu.sync_copy(d0.at[pl.ds(wid * _RPW, _RPW)], d0_v)
    pltpu.sync_copy(d1.at[pl.ds(wid * _RPW, _RPW)], d1_v)
    pltpu.sync_copy(d2.at[pl.ds(wid * _RPW, _RPW)], d2_v)
    pltpu.sync_copy(wb, wb_v)

    def start(j, buf, sem, src):
        pltpu.make_async_copy(src.at[idx_v.at[j]], buf, sem).start()

    def wait(j, buf, sem, src):
        pltpu.make_async_copy(src.at[idx_v.at[j]], buf, sem).wait()

    lanes = lax.iota(jnp.int32, 16)

    def compute(buf, yv, lane_base):
        nv = _TW // 16
        for rr in range(_CH):
            base = rr * _F

            def fbody(f, accs):
                return tuple(a + buf[base + f, pl.ds(v * 16, 16)]
                             for v, a in enumerate(accs))

            accs = lax.fori_loop(
                1, _F, fbody,
                tuple(buf[base, pl.ds(v * 16, 16)] for v in range(nv)))
            sq = accs[0] * accs[0]
            for v in range(1, _D // 16):
                sq = sq + accs[v] * accs[v]
            s = jnp.sum(0.5 * sq + accs[_D // 16])
            yv = jnp.where(lanes == lane_base + rr, s, yv)
        return yv

    bufs = [buf0, buf1, buf2, buf3]
    sems = [sem0, sem1, sem2, sem3]
    srcs = [tspm, taug, tspm, tspm]
    for p in range(4):
        start(p, bufs[p], sems[p], srcs[p])

    w0 = wb_v[0, pl.ds(0, 16)]
    w1 = wb_v[1, pl.ds(0, 16)]
    w2 = wb_v[2, pl.ds(0, 16)]
    bb = wb_v[3, pl.ds(0, 16)]

    def loop_body(i, carry):
        j0 = 4 * i
        yv = jnp.zeros((16,), jnp.float32)
        for p in range(4):
            wait(j0 + p, bufs[p], sems[p], srcs[p])
            yv = compute(bufs[p], yv, 4 * p)
            start(jnp.minimum(j0 + p + 4, _NCH - 1), bufs[p], sems[p],
                  srcs[p])

        sl = pl.ds(i * 16, 16)
        t = w0 * d0_v[sl] + w1 * d1_v[sl] + w2 * d2_v[sl] + bb
        sfull = yv + t
        out_v[sl] = 1.0 / (1.0 + jnp.exp(-sfull))
        return carry

    lax.fori_loop(0, _NCH // 4, loop_body, 0)
    for p in range(4):
        wait(_NCH - 1, bufs[p], sems[p], srcs[p])
    pltpu.sync_copy(out_v, out.at[pl.ds(wid * _RPW, _RPW)])


@jax.jit
def kernel(dense_input, sparse_input, emb1, emb2, W1, b1):
    taug = pl.pallas_call(
        _prep_body,
        out_shape=jax.ShapeDtypeStruct((_VP, _TW), jnp.float32),
    )(emb1, emb2)

    idxp = sparse_input.astype(jnp.int32).reshape(_B // _CH, _IW)
    d0 = dense_input[:, 0]
    d1 = dense_input[:, 1]
    d2 = dense_input[:, 2]
    wb = jnp.broadcast_to(
        jnp.concatenate([W1.reshape(3), b1.reshape(1)]).astype(jnp.float32)[:, None],
        (4, 16),
    )

    mesh = plsc.VectorSubcoreMesh(core_axis_name="c", subcore_axis_name="s")
    sc = functools.partial(
        pl.kernel,
        out_type=jax.ShapeDtypeStruct((_B,), jnp.float32),
        mesh=mesh,
        compiler_params=pltpu.CompilerParams(
            needs_layout_passes=False, use_tc_tiling_on_sc=False
        ),
        scratch_types=[
            pltpu.VMEM((_NCH, _IW), jnp.int32),
            pltpu.VMEM_SHARED((2080, _TW), jnp.float32),
            pltpu.VMEM((_IW, _TW), jnp.float32),
            pltpu.VMEM((_IW, _TW), jnp.float32),
            pltpu.VMEM((_IW, _TW), jnp.float32),
            pltpu.VMEM((_IW, _TW), jnp.float32),
            pltpu.VMEM((_RPW,), jnp.float32),
            pltpu.VMEM((_RPW,), jnp.float32),
            pltpu.VMEM((_RPW,), jnp.float32),
            pltpu.VMEM((4, 16), jnp.float32),
            pltpu.VMEM((_RPW,), jnp.float32),
            pltpu.SemaphoreType.DMA,
            pltpu.SemaphoreType.DMA,
            pltpu.SemaphoreType.DMA,
            pltpu.SemaphoreType.DMA,
        ],
    )(_sc_body)
    y = sc(taug, idxp, d0, d1, d2, wb)
    return y.reshape(_B, 1)


# speedup vs baseline: 1.5024x; 1.0412x over previous
import functools

import jax
import jax.numpy as jnp
from jax import lax
from jax.experimental import pallas as pl
from jax.experimental.pallas import tpu as pltpu
from jax.experimental.pallas import tpu_sc as plsc

_B = 16384
_F = 26
_V = 2068
_D = 128
_TW = 128
_NC = 2
_NS = 16
_NW = _NC * _NS
_RPW = _B // _NW
_CH = 4
_NCH = _RPW // _CH
_IW = _CH * _F


_VP = 2080


def _prep_body(emb1_ref, emb2_ref, taug_ref, c_ref):
    e2 = emb2_ref[...]
    rowsq = jnp.sum(e2 * e2, axis=1, keepdims=True)
    comb = emb1_ref[...] - 0.5 * rowsq
    taug_ref[...] = jnp.concatenate(
        [e2, jnp.zeros((_VP - _V, _TW), jnp.float32)], axis=0)
    c_ref[...] = jnp.concatenate(
        [comb, jnp.zeros((_VP - _V, 1), jnp.float32)], axis=0)


def _sc_body(taug, cvec, idxp, d0, d1, d2, wb, out,
             idx_v, tspm, c_v, buf0, buf1, buf2, buf3, d0_v, d1_v, d2_v,
             wb_v, out_v, sem0, sem1, sem2, sem3):
    wid = lax.axis_index("s") * _NC + lax.axis_index("c")

    sid = lax.axis_index("s")
    rows_per_sub = 130
    rbase = sid * rows_per_sub
    pltpu.sync_copy(taug.at[pl.ds(rbase, rows_per_sub)],
                    tspm.at[pl.ds(rbase, rows_per_sub)])

    plsc.subcore_barrier()

    pltpu.sync_copy(cvec, c_v)
    pltpu.sync_copy(idxp.at[pl.ds(wid * _NCH, _NCH)], idx_v)
    pltpu.sync_copy(d0.at[pl.ds(wid * _RPW, _RPW)], d0_v)
    pltpu.sync_copy(d1.at[pl.ds(wid * _RPW, _RPW)], d1_v)
    pltpu.sync_copy(d2.at[pl.ds(wid * _RPW, _RPW)], d2_v)
    pltpu.sync_copy(wb, wb_v)

    def start(j, buf, sem, src):
        pltpu.make_async_copy(src.at[idx_v.at[j]], buf, sem).start()

    def wait(j, buf, sem, src):
        pltpu.make_async_copy(src.at[idx_v.at[j]], buf, sem).wait()

    lanes = lax.iota(jnp.int32, 16)

    def compute(j, buf, yv, lane_base):
        nv = _TW // 16
        for rr in range(_CH):
            base = rr * _F

            def fbody(f, accs):
                return tuple(a + buf[base + f, pl.ds(v * 16, 16)]
                             for v, a in enumerate(accs))

            accs = lax.fori_loop(
                1, _F, fbody,
                tuple(buf[base, pl.ds(v * 16, 16)] for v in range(nv)))
            sq = accs[0] * accs[0]
            for v in range(1, nv):
                sq = sq + accs[v] * accs[v]
            iv1 = idx_v[j, pl.ds(base, 16)]
            iv2 = idx_v[j, pl.ds(base + 10, 16)]
            g1 = plsc.load_gather(c_v, [iv1])
            g2 = plsc.load_gather(c_v, [iv2])
            gc = g1 + jnp.where(lanes >= 6, g2, 0.0)
            s = jnp.sum(0.5 * sq + gc)
            yv = jnp.where(lanes == lane_base + rr, s, yv)
        return yv

    bufs = [buf0, buf1, buf2, buf3]
    sems = [sem0, sem1, sem2, sem3]
    srcs = [tspm, taug, tspm, tspm]
    for p in range(4):
        start(p, bufs[p], sems[p], srcs[p])

    w0 = wb_v[0, pl.ds(0, 16)]
    w1 = wb_v[1, pl.ds(0, 16)]
    w2 = wb_v[2, pl.ds(0, 16)]
    bb = wb_v[3, pl.ds(0, 16)]

    def loop_body(i, carry):
        j0 = 4 * i
        yv = jnp.zeros((16,), jnp.float32)
        for p in range(4):
            wait(j0 + p, bufs[p], sems[p], srcs[p])
            yv = compute(j0 + p, bufs[p], yv, 4 * p)
            start(jnp.minimum(j0 + p + 4, _NCH - 1), bufs[p], sems[p],
                  srcs[p])

        sl = pl.ds(i * 16, 16)
        t = w0 * d0_v[sl] + w1 * d1_v[sl] + w2 * d2_v[sl] + bb
        sfull = yv + t
        out_v[sl] = 1.0 / (1.0 + jnp.exp(-sfull))
        return carry

    lax.fori_loop(0, _NCH // 4, loop_body, 0)
    for p in range(4):
        wait(_NCH - 1, bufs[p], sems[p], srcs[p])
    pltpu.sync_copy(out_v, out.at[pl.ds(wid * _RPW, _RPW)])


@jax.jit
def kernel(dense_input, sparse_input, emb1, emb2, W1, b1):
    taug, cvec = pl.pallas_call(
        _prep_body,
        out_shape=[jax.ShapeDtypeStruct((_VP, _TW), jnp.float32),
                   jax.ShapeDtypeStruct((_VP, 1), jnp.float32)],
    )(emb1, emb2)
    cvec = cvec.reshape(_VP)

    idxp = sparse_input.astype(jnp.int32).reshape(_B // _CH, _IW)
    d0 = dense_input[:, 0]
    d1 = dense_input[:, 1]
    d2 = dense_input[:, 2]
    wb = jnp.broadcast_to(
        jnp.concatenate([W1.reshape(3), b1.reshape(1)]).astype(jnp.float32)[:, None],
        (4, 16),
    )

    mesh = plsc.VectorSubcoreMesh(core_axis_name="c", subcore_axis_name="s")
    sc = functools.partial(
        pl.kernel,
        out_type=jax.ShapeDtypeStruct((_B,), jnp.float32),
        mesh=mesh,
        compiler_params=pltpu.CompilerParams(
            needs_layout_passes=False, use_tc_tiling_on_sc=False
        ),
        scratch_types=[
            pltpu.VMEM((_NCH, _IW), jnp.int32),
            pltpu.VMEM_SHARED((2080, _TW), jnp.float32),
            pltpu.VMEM((2080,), jnp.float32),
            pltpu.VMEM((_IW, _TW), jnp.float32),
            pltpu.VMEM((_IW, _TW), jnp.float32),
            pltpu.VMEM((_IW, _TW), jnp.float32),
            pltpu.VMEM((_IW, _TW), jnp.float32),
            pltpu.VMEM((_RPW,), jnp.float32),
            pltpu.VMEM((_RPW,), jnp.float32),
            pltpu.VMEM((_RPW,), jnp.float32),
            pltpu.VMEM((4, 16), jnp.float32),
            pltpu.VMEM((_RPW,), jnp.float32),
            pltpu.SemaphoreType.DMA,
            pltpu.SemaphoreType.DMA,
            pltpu.SemaphoreType.DMA,
            pltpu.SemaphoreType.DMA,
        ],
    )(_sc_body)
    y = sc(taug, cvec, idxp, d0, d1, d2, wb)
    return y.reshape(_B, 1)
